# baseline (speedup 1.0000x reference)
import jax
import jax.numpy as jnp
from jax.experimental import pallas as pl
from jax.experimental.pallas import tpu as pltpu

S, B, K, D = 8, 1024, 64, 32
KB = 64
NKB = K // KB


def _select_body(z_ref, mu_ref, sg_ref, p_ref, out_ref, acc_sg, zb):
  kb = pl.program_id(1)

  @pl.when(kb == 0)
  def _bcast():
    zb[...] = jnp.broadcast_to(z_ref[0, 0, :][None, :], (D, B))

  zt = zb[...]
  acc_mu = out_ref[0]
  acc_s = acc_sg[...]
  for kk in range(KB):
    m = zt == (kb * KB + kk)
    acc_mu = jnp.where(m, mu_ref[0, kk], acc_mu)
    acc_s = jnp.where(m, sg_ref[0, kk], acc_s)

  @pl.when(kb < NKB - 1)
  def _store():
    out_ref[0] = acc_mu
    acc_sg[...] = acc_s

  @pl.when(kb == NKB - 1)
  def _fin():
    out_ref[0] = acc_mu + acc_s * p_ref[0]


@jax.jit
def _sample_point_tc(mus_t, sig_t, p_t, z3):
  return pl.pallas_call(
      _select_body,
      grid=(S, NKB),
      in_specs=[
          pl.BlockSpec((1, 1, B), lambda s, kb: (s, 0, 0)),
          pl.BlockSpec((1, KB, D, B), lambda s, kb: (s, kb, 0, 0)),
          pl.BlockSpec((1, KB, D, B), lambda s, kb: (s, kb, 0, 0)),
          pl.BlockSpec((1, D, B), lambda s, kb: (s, 0, 0)),
      ],
      out_specs=pl.BlockSpec((1, D, B), lambda s, kb: (s, 0, 0)),
      out_shape=jax.ShapeDtypeStruct((S, D, B), jnp.float32),
      scratch_shapes=[pltpu.VMEM((D, B), jnp.float32),
                      pltpu.VMEM((D, B), jnp.int32)],
      compiler_params=pltpu.CompilerParams(
          dimension_semantics=("arbitrary", "arbitrary")),
  )(z3, mus_t, sig_t, p_t)


def kernel(p, mus, sigmas, z):
  mus_t = mus.transpose(0, 2, 3, 1)
  sig_t = sigmas.transpose(0, 2, 3, 1)
  p_t = p.transpose(0, 2, 1)
  z3 = z.reshape(S, 1, B).astype(jnp.int32)
  out_t = _sample_point_tc(mus_t, sig_t, p_t, z3)
  return out_t.transpose(0, 2, 1)


# speedup vs baseline: 1.0957x; 1.0957x over previous
import jax
import jax.numpy as jnp
from jax.experimental import pallas as pl
from jax.experimental.pallas import tpu as pltpu

S, B, K, D = 8, 1024, 64, 32
KB = 32
NKB = K // KB


def _select_body(z_ref, mu_ref, sg_ref, p_ref, out_ref, acc_sg, zb):
  kb = pl.program_id(1)

  @pl.when(kb == 0)
  def _bcast():
    zb[...] = jnp.broadcast_to(z_ref[0, 0, :][None, :], (D, B))

  zt = zb[...]
  acc_mu = out_ref[0]
  acc_s = acc_sg[...]
  for kk in range(KB):
    m = zt == (kb * KB + kk)
    acc_mu = jnp.where(m, mu_ref[0, kk], acc_mu)
    acc_s = jnp.where(m, sg_ref[0, kk], acc_s)

  @pl.when(kb < NKB - 1)
  def _store():
    out_ref[0] = acc_mu
    acc_sg[...] = acc_s

  @pl.when(kb == NKB - 1)
  def _fin():
    out_ref[0] = acc_mu + acc_s * p_ref[0]


@jax.jit
def _sample_point_tc(mus_t, sig_t, p_t, z3):
  return pl.pallas_call(
      _select_body,
      grid=(S, NKB),
      in_specs=[
          pl.BlockSpec((1, 1, B), lambda s, kb: (s, 0, 0)),
          pl.BlockSpec((1, KB, D, B), lambda s, kb: (s, kb, 0, 0)),
          pl.BlockSpec((1, KB, D, B), lambda s, kb: (s, kb, 0, 0)),
          pl.BlockSpec((1, D, B), lambda s, kb: (s, 0, 0)),
      ],
      out_specs=pl.BlockSpec((1, D, B), lambda s, kb: (s, 0, 0)),
      out_shape=jax.ShapeDtypeStruct((S, D, B), jnp.float32),
      scratch_shapes=[pltpu.VMEM((D, B), jnp.float32),
                      pltpu.VMEM((D, B), jnp.int32)],
      compiler_params=pltpu.CompilerParams(
          dimension_semantics=("parallel", "arbitrary")),
  )(z3, mus_t, sig_t, p_t)


def kernel(p, mus, sigmas, z):
  mus_t = mus.transpose(0, 2, 3, 1)
  sig_t = sigmas.transpose(0, 2, 3, 1)
  p_t = p.transpose(0, 2, 1)
  z3 = z.reshape(S, 1, B).astype(jnp.int32)
  out_t = _sample_point_tc(mus_t, sig_t, p_t, z3)
  return out_t.transpose(0, 2, 1)
